# baseline (speedup 1.0000x reference)
import functools

import jax
import jax.numpy as jnp
import numpy as np
from jax.experimental import pallas as pl
from jax.experimental.pallas import tpu as pltpu

B_GRAPHS = 32
MAXN = 600
H = 256
HEADS = 4


def _bn(x, eps=1e-5):
    mu = jnp.mean(x, axis=0)
    var = jnp.var(x, axis=0)
    return (x - mu) / jnp.sqrt(var + eps)


def _mish(x):
    return x * jnp.tanh(jax.nn.softplus(x))


def _mlp(x, W, b):
    return _mish(_bn(x @ W + b))


def _rbf(D, D_min=0.0, D_max=6.0, D_count=9):
    mu = jnp.linspace(D_min, D_max, D_count).reshape(1, -1)
    sigma = (D_max - D_min) / D_count
    return jnp.exp(-(((D[:, None] - mu) / sigma) ** 2))


def _gnn_norm(x3, mish=True):
    b, n, c = x3.shape
    x2 = _bn(x3.reshape(-1, c))
    if mish:
        x2 = _mish(x2)
    return x2.reshape(b, n, c)


def _matmul_bias_pallas(x, W, b, block_m=1000):
    M, K = x.shape
    N = W.shape[1]
    b2 = b.reshape(1, N)

    def body(x_ref, w_ref, b_ref, o_ref):
        o_ref[...] = (
            jnp.dot(x_ref[...], w_ref[...], preferred_element_type=jnp.float32)
            + b_ref[...]
        )

    grid = M // block_m
    return pl.pallas_call(
        body,
        grid=(grid,),
        in_specs=[
            pl.BlockSpec((block_m, K), lambda i: (i, 0)),
            pl.BlockSpec((K, N), lambda i: (0, 0)),
            pl.BlockSpec((1, N), lambda i: (0, 0)),
        ],
        out_specs=pl.BlockSpec((block_m, N), lambda i: (i, 0)),
        out_shape=jax.ShapeDtypeStruct((M, N), jnp.float32),
    )(x, W, b2)


def _hil_premlp(rbf_feat, p):
    t = _matmul_bias_pallas(rbf_feat, p['coord_W'], p['coord_b'])
    return _mish(_bn(t))


def _hil_from_radial(x, row, col, radial, p):
    msg = x[row] * radial
    agg = jnp.zeros_like(x).at[col].add(msg)
    t = _matmul_bias_pallas(agg, p['out_W'], p['out_b'])
    return _mish(_bn(t)) + x


def _dense_gcn(x, adj, W, bias, mask=None, improved=True):
    n = adj.shape[1]
    idx = jnp.arange(n)
    adj = adj.at[:, idx, idx].set(2.0 if improved else 1.0)
    out = x @ W
    deg = jnp.maximum(jnp.sum(adj, axis=-1), 1.0) ** -0.5
    adj = deg[:, :, None] * adj * deg[:, None, :]
    out = jnp.matmul(adj, out) + bias
    if mask is not None:
        out = out * mask[:, :, None].astype(x.dtype)
    return out


def _diffpool(xd, mask, adj, p):
    s = _dense_gcn(xd, adj, p['p_W'], p['p_b'], mask)
    s = _gnn_norm(s, mish=True)
    s = jax.nn.softmax(s, axis=-1)
    mf = mask[:, :, None].astype(xd.dtype)
    xdm = xd * mf
    s = s * mf
    out = jnp.matmul(jnp.swapaxes(s, 1, 2), xdm)
    out_adj = jnp.matmul(jnp.matmul(jnp.swapaxes(s, 1, 2), adj), s)
    x2 = _dense_gcn(out, out_adj, p['e_W'], p['e_b'], None)
    x2 = _gnn_norm(x2, mish=True)
    x2 = _gnn_norm(x2 @ p['o_W'] + p['o_b'], mish=False)
    return x2


def _attblock(q, k, v, p):
    res = jnp.sum(q, axis=1)
    bq, sq, _ = q.shape
    sk = k.shape[1]
    hd = H // HEADS
    Q = (q @ p['WQ']).reshape(bq, sq, HEADS, hd).transpose(0, 2, 1, 3)
    K = (k @ p['WK']).reshape(bq, sk, HEADS, hd).transpose(0, 2, 1, 3)
    V = (v @ p['WV']).reshape(bq, sk, HEADS, hd).transpose(0, 2, 1, 3)
    att = jax.nn.softmax(jnp.matmul(Q, jnp.swapaxes(K, -2, -1)) / np.sqrt(hd), axis=-1)
    xo = jnp.matmul(att, V).transpose(0, 2, 1, 3).reshape(bq, sq, H)
    xo = jnp.sum(xo, axis=1)
    return _mlp(xo, p['WO_W'], p['WO_b']) + res


def _to_dense_batch(x, batch, ptr):
    n = x.shape[0]
    pos_in = jnp.arange(n) - ptr[batch]
    dense = jnp.zeros((B_GRAPHS, MAXN, x.shape[1]), x.dtype).at[batch, pos_in].set(x, mode='drop')
    mask = jnp.zeros((B_GRAPHS, MAXN), dtype=bool).at[batch, pos_in].set(True, mode='drop')
    return dense, mask


def _to_dense_adj2(ei, batch, ptr, ew2):
    i0 = batch[ei[0]]
    i1 = ei[0] - ptr[batch[ei[0]]]
    i2 = ei[1] - ptr[batch[ei[1]]]
    return jnp.zeros((B_GRAPHS, MAXN, MAXN, 2), jnp.float32).at[i0, i1, i2].add(ew2, mode='drop')


def kernel(x, pos, params, batch, split, ei_intra, ei_inter):
    counts = jnp.bincount(batch, length=B_GRAPHS)
    ptr = jnp.concatenate([jnp.zeros(1, counts.dtype), jnp.cumsum(counts)])
    lig = split[ei_intra[0]] == 0
    w_lig = lig.astype(jnp.float32)
    w_pro = 1.0 - w_lig

    def edge_geom(ei):
        d = pos[ei[0]] - pos[ei[1]]
        dist = jnp.sqrt(jnp.sum(d * d, axis=-1) + 1e-12)
        return _rbf(dist)

    rbf_intra = edge_geom(ei_intra)
    rbf_inter = edge_geom(ei_inter)

    h = _mish(_bn(_matmul_bias_pallas(x, params['emb']['W'], params['emb']['b'])))

    for ki, ke in (('g1i', 'g1e'), ('g2i', 'g2e'), ('g3i', 'g3e')):
        rad_i = _hil_premlp(rbf_intra, params[ki])
        rad_e = _hil_premlp(rbf_inter, params[ke])
        hi = _hil_from_radial(h, ei_intra[0], ei_intra[1], rad_i, params[ki])
        he = _hil_from_radial(h, ei_inter[0], ei_inter[1], rad_e, params[ke])
        h = (hi + he) / 2.0

    xd, mask = _to_dense_batch(h, batch, ptr)
    adj2 = _to_dense_adj2(ei_intra, batch, ptr, jnp.stack([w_lig, w_pro], axis=-1))
    x_lig = _diffpool(xd, mask, adj2[..., 0], params['dp1'])
    x_pro = _diffpool(xd, mask, adj2[..., 1], params['dp2'])

    z = _attblock(x_lig, x_pro, x_pro, params['att1']) + _attblock(x_pro, x_lig, x_lig, params['att2'])
    z = _mlp(z, params['fc']['W1'], params['fc']['b1'])
    z = z @ params['fc']['W2'] + params['fc']['b2']
    return z.reshape(-1)


# speedup vs baseline: 1.0423x; 1.0423x over previous
import functools

import jax
import jax.numpy as jnp
import numpy as np
from jax.experimental import pallas as pl
from jax.experimental.pallas import tpu as pltpu

B_GRAPHS = 32
MAXN = 600
H = 256
HEADS = 4


def _bn(x, eps=1e-5):
    mu = jnp.mean(x, axis=0)
    var = jnp.var(x, axis=0)
    return (x - mu) / jnp.sqrt(var + eps)


def _mish(x):
    return x * jnp.tanh(jax.nn.softplus(x))


def _mlp(x, W, b):
    return _mish(_bn(x @ W + b))


def _rbf(D, D_min=0.0, D_max=6.0, D_count=9):
    mu = jnp.linspace(D_min, D_max, D_count).reshape(1, -1)
    sigma = (D_max - D_min) / D_count
    return jnp.exp(-(((D[:, None] - mu) / sigma) ** 2))


def _gnn_norm(x3, mish=True):
    b, n, c = x3.shape
    x2 = _bn(x3.reshape(-1, c))
    if mish:
        x2 = _mish(x2)
    return x2.reshape(b, n, c)


def _radial_fused_pallas(rbf_feat, m1, M2, W, b, block_m=2000, eps=1e-5):
    M, K = rbf_feat.shape
    N = W.shape[1]
    mean_t = m1 @ W + b
    e2 = jnp.einsum('kn,kj,jn->n', W, M2, W)
    var_t = e2 - mean_t * mean_t
    scale = (1.0 / jnp.sqrt(var_t + eps)).reshape(1, N)
    shift = mean_t.reshape(1, N)
    b2 = b.reshape(1, N)

    def body(x_ref, w_ref, b_ref, sc_ref, sh_ref, o_ref):
        t = (
            jnp.dot(x_ref[...], w_ref[...], preferred_element_type=jnp.float32)
            + b_ref[...]
        )
        t = (t - sh_ref[...]) * sc_ref[...]
        o_ref[...] = t * jnp.tanh(jax.nn.softplus(t))

    return pl.pallas_call(
        body,
        grid=(M // block_m,),
        in_specs=[
            pl.BlockSpec((block_m, K), lambda i: (i, 0)),
            pl.BlockSpec((K, N), lambda i: (0, 0)),
            pl.BlockSpec((1, N), lambda i: (0, 0)),
            pl.BlockSpec((1, N), lambda i: (0, 0)),
            pl.BlockSpec((1, N), lambda i: (0, 0)),
        ],
        out_specs=pl.BlockSpec((block_m, N), lambda i: (i, 0)),
        out_shape=jax.ShapeDtypeStruct((M, N), jnp.float32),
    )(rbf_feat, W, b2, scale, shift)


def _hil_from_radial(x, row, col, radial, p):
    msg = x[row] * radial
    agg = jnp.zeros_like(x).at[col].add(msg)
    return _mlp(agg, p['out_W'], p['out_b']) + x


def _dense_gcn(x, adj, W, bias, mask=None, improved=True):
    n = adj.shape[1]
    idx = jnp.arange(n)
    adj = adj.at[:, idx, idx].set(2.0 if improved else 1.0)
    out = x @ W
    deg = jnp.maximum(jnp.sum(adj, axis=-1), 1.0) ** -0.5
    adj = deg[:, :, None] * adj * deg[:, None, :]
    out = jnp.matmul(adj, out) + bias
    if mask is not None:
        out = out * mask[:, :, None].astype(x.dtype)
    return out


def _diffpool(xd, mask, adj, p):
    s = _dense_gcn(xd, adj, p['p_W'], p['p_b'], mask)
    s = _gnn_norm(s, mish=True)
    s = jax.nn.softmax(s, axis=-1)
    mf = mask[:, :, None].astype(xd.dtype)
    xdm = xd * mf
    s = s * mf
    out = jnp.matmul(jnp.swapaxes(s, 1, 2), xdm)
    out_adj = jnp.matmul(jnp.matmul(jnp.swapaxes(s, 1, 2), adj), s)
    x2 = _dense_gcn(out, out_adj, p['e_W'], p['e_b'], None)
    x2 = _gnn_norm(x2, mish=True)
    x2 = _gnn_norm(x2 @ p['o_W'] + p['o_b'], mish=False)
    return x2


def _attblock(q, k, v, p):
    res = jnp.sum(q, axis=1)
    bq, sq, _ = q.shape
    sk = k.shape[1]
    hd = H // HEADS
    Q = (q @ p['WQ']).reshape(bq, sq, HEADS, hd).transpose(0, 2, 1, 3)
    K = (k @ p['WK']).reshape(bq, sk, HEADS, hd).transpose(0, 2, 1, 3)
    V = (v @ p['WV']).reshape(bq, sk, HEADS, hd).transpose(0, 2, 1, 3)
    att = jax.nn.softmax(jnp.matmul(Q, jnp.swapaxes(K, -2, -1)) / np.sqrt(hd), axis=-1)
    xo = jnp.matmul(att, V).transpose(0, 2, 1, 3).reshape(bq, sq, H)
    xo = jnp.sum(xo, axis=1)
    return _mlp(xo, p['WO_W'], p['WO_b']) + res


def _to_dense_batch(x, batch, ptr):
    n = x.shape[0]
    pos_in = jnp.arange(n) - ptr[batch]
    dense = jnp.zeros((B_GRAPHS, MAXN, x.shape[1]), x.dtype).at[batch, pos_in].set(x, mode='drop')
    mask = jnp.zeros((B_GRAPHS, MAXN), dtype=bool).at[batch, pos_in].set(True, mode='drop')
    return dense, mask


def _to_dense_adj2(ei, batch, ptr, ew2):
    i0 = batch[ei[0]]
    i1 = ei[0] - ptr[batch[ei[0]]]
    i2 = ei[1] - ptr[batch[ei[1]]]
    return jnp.zeros((B_GRAPHS, MAXN, MAXN, 2), jnp.float32).at[i0, i1, i2].add(ew2, mode='drop')


def kernel(x, pos, params, batch, split, ei_intra, ei_inter):
    counts = jnp.bincount(batch, length=B_GRAPHS)
    ptr = jnp.concatenate([jnp.zeros(1, counts.dtype), jnp.cumsum(counts)])
    lig = split[ei_intra[0]] == 0
    w_lig = lig.astype(jnp.float32)
    w_pro = 1.0 - w_lig

    def edge_geom(ei):
        d = pos[ei[0]] - pos[ei[1]]
        dist = jnp.sqrt(jnp.sum(d * d, axis=-1) + 1e-12)
        return _rbf(dist)

    rbf_intra = edge_geom(ei_intra)
    rbf_inter = edge_geom(ei_inter)
    m1_i = jnp.mean(rbf_intra, axis=0)
    M2_i = (rbf_intra.T @ rbf_intra) / rbf_intra.shape[0]
    m1_e = jnp.mean(rbf_inter, axis=0)
    M2_e = (rbf_inter.T @ rbf_inter) / rbf_inter.shape[0]

    h = _mlp(x, params['emb']['W'], params['emb']['b'])

    for ki, ke in (('g1i', 'g1e'), ('g2i', 'g2e'), ('g3i', 'g3e')):
        rad_i = _radial_fused_pallas(rbf_intra, m1_i, M2_i, params[ki]['coord_W'], params[ki]['coord_b'])
        rad_e = _radial_fused_pallas(rbf_inter, m1_e, M2_e, params[ke]['coord_W'], params[ke]['coord_b'])
        hi = _hil_from_radial(h, ei_intra[0], ei_intra[1], rad_i, params[ki])
        he = _hil_from_radial(h, ei_inter[0], ei_inter[1], rad_e, params[ke])
        h = (hi + he) / 2.0

    xd, mask = _to_dense_batch(h, batch, ptr)
    adj2 = _to_dense_adj2(ei_intra, batch, ptr, jnp.stack([w_lig, w_pro], axis=-1))
    x_lig = _diffpool(xd, mask, adj2[..., 0], params['dp1'])
    x_pro = _diffpool(xd, mask, adj2[..., 1], params['dp2'])

    z = _attblock(x_lig, x_pro, x_pro, params['att1']) + _attblock(x_pro, x_lig, x_lig, params['att2'])
    z = _mlp(z, params['fc']['W1'], params['fc']['b1'])
    z = z @ params['fc']['W2'] + params['fc']['b2']
    return z.reshape(-1)


# speedup vs baseline: 1.0572x; 1.0144x over previous
import jax
import jax.numpy as jnp
import numpy as np
from jax.experimental import pallas as pl

B_GRAPHS = 32
MAXN = 600
H = 256
HEADS = 4

def _bn(x, eps=1e-5):
    mu = jnp.mean(x, axis=0)
    var = jnp.var(x, axis=0)
    return (x - mu) / jnp.sqrt(var + eps)


def _mish(x):
    return x * jnp.tanh(jax.nn.softplus(x))


def _mlp(x, W, b):
    return _mish(_bn(x @ W + b))


def _rbf(D, D_min=0.0, D_max=6.0, D_count=9):
    mu = jnp.linspace(D_min, D_max, D_count).reshape(1, -1)
    sigma = (D_max - D_min) / D_count
    return jnp.exp(-(((D[:, None] - mu) / sigma) ** 2))


def _gnn_norm(x3, mish=True):
    b, n, c = x3.shape
    x2 = _bn(x3.reshape(-1, c))
    if mish:
        x2 = _mish(x2)
    return x2.reshape(b, n, c)


def _radial_fused_pallas(rbf_feat, m1, M2, W, b, block_m=2000, eps=1e-5):
    M, K = rbf_feat.shape
    N = W.shape[1]
    mean_t = m1 @ W + b
    e2 = jnp.einsum('kn,kj,jn->n', W, M2, W)
    var_t = e2 - mean_t * mean_t
    scale = (1.0 / jnp.sqrt(var_t + eps)).reshape(1, N)
    shift = mean_t.reshape(1, N)
    b2 = b.reshape(1, N)

    def body(x_ref, w_ref, b_ref, sc_ref, sh_ref, o_ref):
        t = (
            jnp.dot(x_ref[...], w_ref[...], preferred_element_type=jnp.float32)
            + b_ref[...]
        )
        t = (t - sh_ref[...]) * sc_ref[...]
        o_ref[...] = (t * jnp.tanh(jax.nn.softplus(t))).astype(jnp.bfloat16)

    return pl.pallas_call(
        body,
        grid=(M // block_m,),
        in_specs=[
            pl.BlockSpec((block_m, K), lambda i: (i, 0)),
            pl.BlockSpec((K, N), lambda i: (0, 0)),
            pl.BlockSpec((1, N), lambda i: (0, 0)),
            pl.BlockSpec((1, N), lambda i: (0, 0)),
            pl.BlockSpec((1, N), lambda i: (0, 0)),
        ],
        out_specs=pl.BlockSpec((block_m, N), lambda i: (i, 0)),
        out_shape=jax.ShapeDtypeStruct((M, N), jnp.bfloat16),
    )(rbf_feat, W, b2, scale, shift)


def _hil_from_radial(x, row, col, radial, p):
    msg = x[row] * radial.astype(jnp.float32)
    agg = jnp.zeros_like(x).at[col].add(msg)
    return _mlp(agg, p['out_W'], p['out_b']) + x


def _dense_gcn(x, adj, W, bias, mask=None, improved=True):
    n = adj.shape[1]
    idx = jnp.arange(n)
    adj = adj.at[:, idx, idx].set(2.0 if improved else 1.0)
    out = x @ W
    deg = jnp.maximum(jnp.sum(adj, axis=-1), 1.0) ** -0.5
    adj = deg[:, :, None] * adj * deg[:, None, :]
    out = jnp.matmul(adj, out) + bias
    if mask is not None:
        out = out * mask[:, :, None].astype(x.dtype)
    return out


def _diffpool(xd, mask, adj, p):
    s = _dense_gcn(xd, adj, p['p_W'], p['p_b'], mask)
    s = _gnn_norm(s, mish=True)
    s = jax.nn.softmax(s, axis=-1)
    mf = mask[:, :, None].astype(xd.dtype)
    xdm = xd * mf
    s = s * mf
    out = jnp.matmul(jnp.swapaxes(s, 1, 2), xdm)
    out_adj = jnp.matmul(jnp.matmul(jnp.swapaxes(s, 1, 2), adj), s)
    x2 = _dense_gcn(out, out_adj, p['e_W'], p['e_b'], None)
    x2 = _gnn_norm(x2, mish=True)
    x2 = _gnn_norm(x2 @ p['o_W'] + p['o_b'], mish=False)
    return x2


def _attblock(q, k, v, p):
    res = jnp.sum(q, axis=1)
    bq, sq, _ = q.shape
    sk = k.shape[1]
    hd = H // HEADS
    Q = (q @ p['WQ']).reshape(bq, sq, HEADS, hd).transpose(0, 2, 1, 3)
    K = (k @ p['WK']).reshape(bq, sk, HEADS, hd).transpose(0, 2, 1, 3)
    V = (v @ p['WV']).reshape(bq, sk, HEADS, hd).transpose(0, 2, 1, 3)
    att = jax.nn.softmax(jnp.matmul(Q, jnp.swapaxes(K, -2, -1)) / np.sqrt(hd), axis=-1)
    xo = jnp.matmul(att, V).transpose(0, 2, 1, 3).reshape(bq, sq, H)
    xo = jnp.sum(xo, axis=1)
    return _mlp(xo, p['WO_W'], p['WO_b']) + res


def _to_dense_batch(x, batch, ptr):
    n = x.shape[0]
    pos_in = jnp.arange(n) - ptr[batch]
    dense = jnp.zeros((B_GRAPHS, MAXN, x.shape[1]), x.dtype).at[batch, pos_in].set(x, mode='drop')
    mask = jnp.zeros((B_GRAPHS, MAXN), dtype=bool).at[batch, pos_in].set(True, mode='drop')
    return dense, mask


def _to_dense_adj2(ei, batch, ptr, ew2):
    i0 = batch[ei[0]]
    i1 = ei[0] - ptr[batch[ei[0]]]
    i2 = ei[1] - ptr[batch[ei[1]]]
    return jnp.zeros((B_GRAPHS, MAXN, MAXN, 2), jnp.float32).at[i0, i1, i2].add(ew2, mode='drop')


def kernel(x, pos, params, batch, split, ei_intra, ei_inter):
    counts = jnp.bincount(batch, length=B_GRAPHS)
    ptr = jnp.concatenate([jnp.zeros(1, counts.dtype), jnp.cumsum(counts)])
    lig = split[ei_intra[0]] == 0
    w_lig = lig.astype(jnp.float32)
    w_pro = 1.0 - w_lig

    def edge_geom(ei):
        d = pos[ei[0]] - pos[ei[1]]
        dist = jnp.sqrt(jnp.sum(d * d, axis=-1) + 1e-12)
        return _rbf(dist)

    rbf_intra = edge_geom(ei_intra)
    rbf_inter = edge_geom(ei_inter)
    m1_i = jnp.mean(rbf_intra, axis=0)
    M2_i = (rbf_intra.T @ rbf_intra) / rbf_intra.shape[0]
    m1_e = jnp.mean(rbf_inter, axis=0)
    M2_e = (rbf_inter.T @ rbf_inter) / rbf_inter.shape[0]

    h = _mlp(x, params['emb']['W'], params['emb']['b'])

    for ki, ke in (('g1i', 'g1e'), ('g2i', 'g2e'), ('g3i', 'g3e')):
        rad_i = _radial_fused_pallas(rbf_intra, m1_i, M2_i, params[ki]['coord_W'], params[ki]['coord_b'])
        rad_e = _radial_fused_pallas(rbf_inter, m1_e, M2_e, params[ke]['coord_W'], params[ke]['coord_b'])
        hi = _hil_from_radial(h, ei_intra[0], ei_intra[1], rad_i, params[ki])
        he = _hil_from_radial(h, ei_inter[0], ei_inter[1], rad_e, params[ke])
        h = (hi + he) / 2.0

    xd, mask = _to_dense_batch(h, batch, ptr)
    adj2 = _to_dense_adj2(ei_intra, batch, ptr, jnp.stack([w_lig, w_pro], axis=-1))
    x_lig = _diffpool(xd, mask, adj2[..., 0], params['dp1'])
    x_pro = _diffpool(xd, mask, adj2[..., 1], params['dp2'])

    z = _attblock(x_lig, x_pro, x_pro, params['att1']) + _attblock(x_pro, x_lig, x_lig, params['att2'])
    z = _mlp(z, params['fc']['W1'], params['fc']['b1'])
    z = z @ params['fc']['W2'] + params['fc']['b2']
    return z.reshape(-1)


# speedup vs baseline: 1.0867x; 1.0278x over previous
import jax
import jax.numpy as jnp
import numpy as np
from jax.experimental import pallas as pl

B_GRAPHS = 32
MAXN = 600
H = 256
HEADS = 4

def _bn(x, eps=1e-5):
    mu = jnp.mean(x, axis=0)
    var = jnp.var(x, axis=0)
    return (x - mu) / jnp.sqrt(var + eps)


def _mish(x):
    return x * jnp.tanh(jax.nn.softplus(x))


def _mlp(x, W, b):
    return _mish(_bn(x @ W + b))


def _rbf(D, D_min=0.0, D_max=6.0, D_count=9):
    mu = jnp.linspace(D_min, D_max, D_count).reshape(1, -1)
    sigma = (D_max - D_min) / D_count
    return jnp.exp(-(((D[:, None] - mu) / sigma) ** 2))


def _gnn_norm(x3, mish=True):
    b, n, c = x3.shape
    x2 = _bn(x3.reshape(-1, c))
    if mish:
        x2 = _mish(x2)
    return x2.reshape(b, n, c)


def _radial_fused_pallas(rbf_feat, m1, M2, W, b, block_m=2000, eps=1e-5):
    M, K = rbf_feat.shape
    N = W.shape[1]
    mean_t = m1 @ W + b
    e2 = jnp.einsum('kn,kj,jn->n', W, M2, W)
    var_t = e2 - mean_t * mean_t
    scale = (1.0 / jnp.sqrt(var_t + eps)).reshape(1, N)
    shift = mean_t.reshape(1, N)
    b2 = b.reshape(1, N)

    def body(x_ref, w_ref, b_ref, sc_ref, sh_ref, o_ref):
        t = (
            jnp.dot(x_ref[...], w_ref[...], preferred_element_type=jnp.float32)
            + b_ref[...]
        )
        t = (t - sh_ref[...]) * sc_ref[...]
        o_ref[...] = t * jnp.tanh(jax.nn.softplus(t))

    return pl.pallas_call(
        body,
        grid=(M // block_m,),
        in_specs=[
            pl.BlockSpec((block_m, K), lambda i: (i, 0)),
            pl.BlockSpec((K, N), lambda i: (0, 0)),
            pl.BlockSpec((1, N), lambda i: (0, 0)),
            pl.BlockSpec((1, N), lambda i: (0, 0)),
            pl.BlockSpec((1, N), lambda i: (0, 0)),
        ],
        out_specs=pl.BlockSpec((block_m, N), lambda i: (i, 0)),
        out_shape=jax.ShapeDtypeStruct((M, N), jnp.float32),
    )(rbf_feat, W, b2, scale, shift)


def _hil_from_radial(x, row, col, radial, p):
    msg = x[row] * radial
    agg = jnp.zeros_like(x).at[col].add(msg)
    return _mlp(agg, p['out_W'], p['out_b']) + x


def _dense_gcn(x, adj, W, bias, mask=None, improved=True):
    n = adj.shape[1]
    idx = jnp.arange(n)
    adj = adj.at[:, idx, idx].set(2.0 if improved else 1.0)
    out = x @ W
    deg = jnp.maximum(jnp.sum(adj, axis=-1), 1.0) ** -0.5
    adj = deg[:, :, None] * adj * deg[:, None, :]
    out = jnp.matmul(adj, out) + bias
    if mask is not None:
        out = out * mask[:, :, None].astype(x.dtype)
    return out


def _diffpool(xd, mask, adj, p):
    s = _dense_gcn(xd, adj, p['p_W'], p['p_b'], mask)
    s = _gnn_norm(s, mish=True)
    s = jax.nn.softmax(s, axis=-1)
    mf = mask[:, :, None].astype(xd.dtype)
    xdm = xd * mf
    s = s * mf
    out = jnp.matmul(jnp.swapaxes(s, 1, 2), xdm)
    out_adj = jnp.matmul(jnp.matmul(jnp.swapaxes(s, 1, 2), adj), s)
    x2 = _dense_gcn(out, out_adj, p['e_W'], p['e_b'], None)
    x2 = _gnn_norm(x2, mish=True)
    x2 = _gnn_norm(x2 @ p['o_W'] + p['o_b'], mish=False)
    return x2


def _attblock(q, k, v, p):
    res = jnp.sum(q, axis=1)
    bq, sq, _ = q.shape
    sk = k.shape[1]
    hd = H // HEADS
    Q = (q @ p['WQ']).reshape(bq, sq, HEADS, hd).transpose(0, 2, 1, 3)
    K = (k @ p['WK']).reshape(bq, sk, HEADS, hd).transpose(0, 2, 1, 3)
    V = (v @ p['WV']).reshape(bq, sk, HEADS, hd).transpose(0, 2, 1, 3)
    att = jax.nn.softmax(jnp.matmul(Q, jnp.swapaxes(K, -2, -1)) / np.sqrt(hd), axis=-1)
    xo = jnp.matmul(att, V).transpose(0, 2, 1, 3).reshape(bq, sq, H)
    xo = jnp.sum(xo, axis=1)
    return _mlp(xo, p['WO_W'], p['WO_b']) + res


def _to_dense_batch(x, batch, ptr):
    n = x.shape[0]
    pos_in = jnp.arange(n) - ptr[batch]
    dense = jnp.zeros((B_GRAPHS, MAXN, x.shape[1]), x.dtype).at[batch, pos_in].set(x, mode='drop')
    mask = jnp.zeros((B_GRAPHS, MAXN), dtype=bool).at[batch, pos_in].set(True, mode='drop')
    return dense, mask


def _to_dense_adj(ei, batch, ptr, ew):
    i0 = batch[ei[0]]
    i1 = ei[0] - ptr[batch[ei[0]]]
    i2 = ei[1] - ptr[batch[ei[1]]]
    return jnp.zeros((B_GRAPHS, MAXN, MAXN), jnp.float32).at[i0, i1, i2].add(ew, mode='drop')


def kernel(x, pos, params, batch, split, ei_intra, ei_inter):
    counts = jnp.bincount(batch, length=B_GRAPHS)
    ptr = jnp.concatenate([jnp.zeros(1, counts.dtype), jnp.cumsum(counts)])
    lig = split[ei_intra[0]] == 0
    w_lig = lig.astype(jnp.float32)
    w_pro = 1.0 - w_lig

    def edge_geom(ei):
        d = pos[ei[0]] - pos[ei[1]]
        dist = jnp.sqrt(jnp.sum(d * d, axis=-1) + 1e-12)
        return _rbf(dist)

    rbf_intra = edge_geom(ei_intra)
    rbf_inter = edge_geom(ei_inter)
    m1_i = jnp.mean(rbf_intra, axis=0)
    M2_i = (rbf_intra.T @ rbf_intra) / rbf_intra.shape[0]
    m1_e = jnp.mean(rbf_inter, axis=0)
    M2_e = (rbf_inter.T @ rbf_inter) / rbf_inter.shape[0]

    h = _mlp(x, params['emb']['W'], params['emb']['b'])

    for ki, ke in (('g1i', 'g1e'), ('g2i', 'g2e'), ('g3i', 'g3e')):
        rad_i = _radial_fused_pallas(rbf_intra, m1_i, M2_i, params[ki]['coord_W'], params[ki]['coord_b'])
        rad_e = _radial_fused_pallas(rbf_inter, m1_e, M2_e, params[ke]['coord_W'], params[ke]['coord_b'])
        hi = _hil_from_radial(h, ei_intra[0], ei_intra[1], rad_i, params[ki])
        he = _hil_from_radial(h, ei_inter[0], ei_inter[1], rad_e, params[ke])
        h = (hi + he) / 2.0

    xd, mask = _to_dense_batch(h, batch, ptr)
    adj_lig = _to_dense_adj(ei_intra, batch, ptr, w_lig)
    adj_pro = _to_dense_adj(ei_intra, batch, ptr, w_pro)
    x_lig = _diffpool(xd, mask, adj_lig, params['dp1'])
    x_pro = _diffpool(xd, mask, adj_pro, params['dp2'])

    z = _attblock(x_lig, x_pro, x_pro, params['att1']) + _attblock(x_pro, x_lig, x_lig, params['att2'])
    z = _mlp(z, params['fc']['W1'], params['fc']['b1'])
    z = z @ params['fc']['W2'] + params['fc']['b2']
    return z.reshape(-1)
